# baseline (speedup 1.0000x reference)
import functools

import jax
import jax.numpy as jnp
from jax import lax
from jax.experimental import pallas as pl
from jax.experimental.pallas import tpu as pltpu
from jax.experimental.pallas import tpu_sc as plsc

NC = 2
NS = 16
NW = NC * NS
C = 120


def _mm_body(x_ref, w_ref, o_ref):
    o_ref[...] = jnp.dot(x_ref[...], w_ref[...],
                         preferred_element_type=jnp.float32)


def _tc_matmul(x, w):
    return pl.pallas_call(
        _mm_body,
        out_shape=jax.ShapeDtypeStruct((x.shape[0], w.shape[1]), jnp.float32),
    )(x, w)


def _mid_body(acc_ref, den_ref, x_ref, s8_ref, b1_ref, bnw_ref, bnb_ref,
              wskt_ref, bsk_ref, wc2t_ref, o_ref):
    acc = acc_ref[0] + acc_ref[1]
    den = den_ref[0] + den_ref[1]
    denrep = jnp.dot(den, s8_ref[...],
                     preferred_element_type=jnp.float32)
    h = acc / denrep + b1_ref[...]
    n = h.shape[0]
    mu = jnp.sum(h, axis=0, keepdims=True) / n
    hc = h - mu
    var = jnp.sum(hc * hc, axis=0, keepdims=True) / n
    h = hc * lax.rsqrt(var + 1e-5) * bnw_ref[...] + bnb_ref[...]
    h = jnp.where(h > 0, h, jnp.exp(jnp.minimum(h, 0.0)) - 1.0)
    hsk = jnp.dot(x_ref[...], wskt_ref[...],
                  preferred_element_type=jnp.float32) + bsk_ref[...] + h
    o_ref[...] = jnp.dot(hsk, wc2t_ref[...],
                         preferred_element_type=jnp.float32)


def _emb_body(acc_ref, den_ref, b2_ref, o_ref):
    acc = acc_ref[0] + acc_ref[1]
    den = den_ref[0] + den_ref[1]
    o_ref[...] = acc / den + b2_ref[...]


def _gelu(z):
    return 0.5 * z * (1.0 + lax.erf(z * 0.7071067811865476))


def _heads_body(e_ref, w1_ref, b1_ref, g_ref, bb_ref, w2_ref, b2_ref,
                w3_ref, b3_ref, o_ref):
    cols = []
    for g in range(3):
        z = jnp.dot(e_ref[...], w1_ref[g],
                    preferred_element_type=jnp.float32) + b1_ref[g]
        m = jnp.mean(z, axis=1, keepdims=True)
        zc = z - m
        v = jnp.mean(zc * zc, axis=1, keepdims=True)
        z = zc * lax.rsqrt(v + 1e-5) * g_ref[g] + bb_ref[g]
        z = _gelu(z)
        z = _gelu(jnp.dot(z, w2_ref[g],
                          preferred_element_type=jnp.float32) + b2_ref[g])
        z = jnp.dot(z, w3_ref[g],
                    preferred_element_type=jnp.float32) + b3_ref[g]
        cols.append(z)
    o_ref[...] = jnp.concatenate(cols, axis=1)


def _edge_kernel(n1, epw, nchunks, smap):
    rpt = n1 // NS
    mesh = plsc.VectorSubcoreMesh(core_axis_name="c", subcore_axis_name="s")

    @functools.partial(
        pl.kernel,
        out_type=jax.ShapeDtypeStruct((NC, n1, 144), jnp.float32),
        mesh=mesh,
        compiler_params=pltpu.CompilerParams(use_tc_tiling_on_sc=False),
        scratch_types=[
            pltpu.VMEM((4, C), jnp.int32),
            pltpu.VMEM((4, C), jnp.int32),
            pltpu.VMEM((2, C, 144), jnp.float32),
            pltpu.VMEM((2, C, 16), jnp.float32),
            pltpu.VMEM_SHARED((n1, 144), jnp.float32),
            [pltpu.SemaphoreType.DMA] * 2,
            [pltpu.SemaphoreType.DMA] * 4,
            [pltpu.SemaphoreType.DMA] * 2,
        ],
    )
    def k(yt, ald, src, dst, zacc, accden_out,
          src_v, dst_v, rows, aldr, accden_s, sems, isems, ssems):
        cid = lax.axis_index("c")
        sid = lax.axis_index("s")
        wid = cid * NS + sid
        pltpu.sync_copy(zacc, accden_s.at[pl.ds(sid * rpt, rpt)])
        plsc.subcore_barrier()

        base = wid * epw

        def fetch_idx(t, q):
            off = base + t * C
            pltpu.async_copy(src.at[pl.ds(off, C)], src_v.at[q], isems[q])
            pltpu.async_copy(dst.at[pl.ds(off, C)], dst_v.at[q], isems[q])

        def drain_idx(q):
            pltpu.make_async_copy(src.at[pl.ds(0, C)], src_v.at[q],
                                  isems[q]).wait()
            pltpu.make_async_copy(dst.at[pl.ds(0, C)], dst_v.at[q],
                                  isems[q]).wait()

        def fetch_rows(q, b):
            pltpu.async_copy(yt.at[src_v.at[q]], rows.at[b], sems[b])
            pltpu.async_copy(ald.at[dst_v.at[q]], aldr.at[b], sems[b])

        def drain_rows(b):
            pltpu.make_async_copy(yt.at[src_v.at[0]], rows.at[b],
                                  sems[b]).wait()
            pltpu.make_async_copy(ald.at[dst_v.at[0]], aldr.at[b],
                                  sems[b]).wait()

        def wait_scatter(b):
            pltpu.make_async_copy(rows.at[b], accden_s.at[dst_v.at[0]],
                                  ssems[b]).wait()

        fetch_idx(0, 0)
        drain_idx(0)
        fetch_rows(0, 0)
        fetch_idx(1, 1)
        fetch_idx(2, 2)

        def quad(tt, carry):
            for i in range(4):
                t = tt * 4 + i
                b = i % 2
                qn = (i + 1) % 4
                drain_rows(b)
                drain_idx(qn)
                if i == 0:
                    @pl.when(tt > 0)
                    def _():
                        wait_scatter(1 - b)
                else:
                    wait_scatter(1 - b)
                fetch_rows(qn, 1 - b)
                fetch_idx(jnp.minimum(t + 3, nchunks - 1), (i + 3) % 4)

                def edge(j, carry2):
                    t9 = rows[b, j, pl.ds(128, 16)] + aldr[b, j, :]
                    t9 = jnp.maximum(t9, t9 * 0.2)
                    ex = jnp.exp(t9)
                    rows[b, j, pl.ds(128, 16)] = ex
                    for kk in range(8):
                        s = ex[smap[kk]]
                        rows[b, j, pl.ds(kk * 16, 16)] = (
                            rows[b, j, pl.ds(kk * 16, 16)] * s)
                    return carry2

                lax.fori_loop(0, C, edge, 0, unroll=12)
                pltpu.async_copy(rows.at[b], accden_s.at[dst_v.at[i]],
                                 ssems[b], add=True)
            return carry

        lax.fori_loop(0, nchunks // 4, quad, 0)
        wait_scatter(1)
        drain_rows(0)
        drain_idx(1)
        drain_idx(2)
        plsc.subcore_barrier()
        pltpu.sync_copy(accden_s.at[pl.ds(sid * rpt, rpt)],
                        accden_out.at[cid, pl.ds(sid * rpt, rpt)])

    return k


def _gather_kernel(n, nrows):
    bpw = nrows // NW
    mesh = plsc.VectorSubcoreMesh(core_axis_name="c", subcore_axis_name="s")

    @functools.partial(
        pl.kernel,
        out_type=jax.ShapeDtypeStruct((nrows, 128), jnp.float32),
        mesh=mesh,
        scratch_types=[
            pltpu.VMEM((bpw,), jnp.int32),
            pltpu.VMEM((bpw, 128), jnp.float32),
            pltpu.SemaphoreType.DMA,
        ],
    )
    def k(emb, uv, out, idx_v, rows_v, sem):
        wid = lax.axis_index("c") * NS + lax.axis_index("s")
        base = wid * bpw
        pltpu.sync_copy(uv.at[pl.ds(base, bpw)], idx_v)
        pltpu.async_copy(emb.at[idx_v], rows_v, sem).wait()
        pltpu.sync_copy(rows_v, out.at[pl.ds(base, bpw)])

    return k


def kernel(x, edge_index, edges, groups, W1, a1_src, a1_dst, b1, bn_w, bn_b,
           Wsk, bsk, W2, a2_src, a2_dst, b2, Hw1, Hb1, Hln_g, Hln_b,
           Hw2, Hb2, Hw3, Hb3):
    f32 = jnp.float32
    N, D = x.shape
    H = W1.shape[0]
    E = edge_index.shape[1]
    B = edges.shape[0]
    heads = a1_src.shape[0]
    oc1 = H // heads

    n1 = ((N + 1 + NS * 8 - 1) // (NS * 8)) * (NS * 8)
    etot = E + N
    epad = ((etot + 4 * NW * C - 1) // (4 * NW * C)) * (4 * NW * C)
    epw = epad // NW
    nchunks = epw // C

    sl = jnp.arange(N, dtype=jnp.int32)
    pad = jnp.full((epad - etot,), N, dtype=jnp.int32)
    srcp = jnp.concatenate([edge_index[0].astype(jnp.int32), sl, pad])
    dstp = jnp.concatenate([edge_index[1].astype(jnp.int32), sl, pad])

    W1r = W1.reshape(heads, oc1, D)
    v1s = jnp.einsum('hc,hcd->hd', a1_src, W1r)
    v1d = jnp.einsum('hc,hcd->hd', a1_dst, W1r)
    wc1t = jnp.concatenate([W1, v1s, v1d], axis=0).T

    v2s = a2_src @ W2
    v2d = a2_dst @ W2
    wc2t = jnp.concatenate(
        [W2, v2s, v2d, jnp.zeros((14, H), f32)], axis=0).T

    zacc = jnp.zeros((n1 // NS, 144), f32)

    y1 = _tc_matmul(x, wc1t)
    y1p = jnp.zeros((n1, 144), f32).at[:N].set(y1)
    ald1 = jnp.zeros((n1, 16), f32).at[:N, :heads].set(y1[:, 136:144])

    ek1 = _edge_kernel(n1, epw, nchunks, list(range(8)))
    ad1 = ek1(y1p, ald1, srcp, dstp, zacc)
    acc1, den1 = ad1[..., :128], ad1[..., 128:]

    s8 = (jnp.arange(128, dtype=jnp.int32)[None, :] // oc1
          == jnp.arange(heads, dtype=jnp.int32)[:, None]).astype(f32)
    y2 = pl.pallas_call(
        _mid_body,
        out_shape=jax.ShapeDtypeStruct((N, 144), f32),
    )(acc1[:, :N], den1[:, :N, :heads], x, s8, b1.reshape(1, H),
      bn_w.reshape(1, H), bn_b.reshape(1, H), Wsk.T, bsk.reshape(1, H),
      wc2t)

    y2p = jnp.zeros((n1, 144), f32).at[:N].set(y2)
    ald2 = jnp.zeros((n1, 16), f32).at[:N, 0].set(y2[:, 129])

    ek2 = _edge_kernel(n1, epw, nchunks, [0] * 8)
    ad2 = ek2(y2p, ald2, srcp, dstp, zacc)
    acc2, den2 = ad2[..., :128], ad2[..., 128:]

    emb = pl.pallas_call(
        _emb_body,
        out_shape=jax.ShapeDtypeStruct((N, H), f32),
    )(acc2[:, :N], den2[:, :N, :1], b2.reshape(1, H))

    uv = jnp.concatenate([edges[:, 0], edges[:, 1]]).astype(jnp.int32)
    euv = _gather_kernel(N, 2 * B)(emb, uv)
    e = jnp.concatenate([euv[:B], euv[B:]], axis=1)

    out3 = pl.pallas_call(
        _heads_body,
        out_shape=jax.ShapeDtypeStruct((B, 3), f32),
    )(e, jnp.transpose(Hw1, (0, 2, 1)), Hb1[:, None, :], Hln_g[:, None, :],
      Hln_b[:, None, :], jnp.transpose(Hw2, (0, 2, 1)), Hb2[:, None, :],
      jnp.transpose(Hw3, (0, 2, 1)), Hb3[:, None, :])

    sel = out3[jnp.arange(B), groups]
    order = jnp.argsort(groups)
    return sel[order]


# speedup vs baseline: 1.4197x; 1.4197x over previous
import functools

import jax
import jax.numpy as jnp
from jax import lax
from jax.experimental import pallas as pl
from jax.experimental.pallas import tpu as pltpu
from jax.experimental.pallas import tpu_sc as plsc

NC = 2
NS = 16
NW = NC * NS
C = 120


def _mm_body(x_ref, w_ref, o_ref):
    o_ref[...] = jnp.dot(x_ref[...], w_ref[...],
                         preferred_element_type=jnp.float32)


def _tc_matmul(x, w):
    return pl.pallas_call(
        _mm_body,
        out_shape=jax.ShapeDtypeStruct((x.shape[0], w.shape[1]), jnp.float32),
    )(x, w)


def _mid_body(acc_ref, den_ref, x_ref, s8_ref, b1_ref, bnw_ref, bnb_ref,
              wskt_ref, bsk_ref, wc2t_ref, o_ref):
    acc = acc_ref[0] + acc_ref[1]
    den = den_ref[0] + den_ref[1]
    denrep = jnp.dot(den, s8_ref[...],
                     preferred_element_type=jnp.float32)
    h = acc / denrep + b1_ref[...]
    n = h.shape[0]
    mu = jnp.sum(h, axis=0, keepdims=True) / n
    hc = h - mu
    var = jnp.sum(hc * hc, axis=0, keepdims=True) / n
    h = hc * lax.rsqrt(var + 1e-5) * bnw_ref[...] + bnb_ref[...]
    h = jnp.where(h > 0, h, jnp.exp(jnp.minimum(h, 0.0)) - 1.0)
    hsk = jnp.dot(x_ref[...], wskt_ref[...],
                  preferred_element_type=jnp.float32) + bsk_ref[...] + h
    o_ref[...] = jnp.dot(hsk, wc2t_ref[...],
                         preferred_element_type=jnp.float32)


def _emb_body(acc_ref, den_ref, b2_ref, o_ref):
    acc = acc_ref[0] + acc_ref[1]
    den = den_ref[0] + den_ref[1]
    o_ref[...] = acc / den + b2_ref[...]


def _gelu(z):
    return 0.5 * z * (1.0 + lax.erf(z * 0.7071067811865476))


def _heads_body(e_ref, w1_ref, b1_ref, g_ref, bb_ref, w2_ref, b2_ref,
                w3_ref, b3_ref, o_ref):
    cols = []
    for g in range(3):
        z = jnp.dot(e_ref[...], w1_ref[g],
                    preferred_element_type=jnp.float32) + b1_ref[g]
        m = jnp.mean(z, axis=1, keepdims=True)
        zc = z - m
        v = jnp.mean(zc * zc, axis=1, keepdims=True)
        z = zc * lax.rsqrt(v + 1e-5) * g_ref[g] + bb_ref[g]
        z = _gelu(z)
        z = _gelu(jnp.dot(z, w2_ref[g],
                          preferred_element_type=jnp.float32) + b2_ref[g])
        z = jnp.dot(z, w3_ref[g],
                    preferred_element_type=jnp.float32) + b3_ref[g]
        cols.append(z)
    o_ref[...] = jnp.concatenate(cols, axis=1)


def _edge_kernel(n1, epw, nchunks, smap):
    rpt = n1 // NS
    mesh = plsc.VectorSubcoreMesh(core_axis_name="c", subcore_axis_name="s")

    @functools.partial(
        pl.kernel,
        out_type=jax.ShapeDtypeStruct((NC, n1, 144), jnp.float32),
        mesh=mesh,
        compiler_params=pltpu.CompilerParams(use_tc_tiling_on_sc=False),
        scratch_types=[
            pltpu.VMEM((2, C), jnp.int32),
            pltpu.VMEM((2, C), jnp.int32),
            pltpu.VMEM((2, C, 144), jnp.float32),
            pltpu.VMEM((2, C, 16), jnp.float32),
            pltpu.VMEM_SHARED((n1, 144), jnp.float32),
            pltpu.SemaphoreType.DMA,
            pltpu.SemaphoreType.DMA,
            pltpu.SemaphoreType.DMA,
            pltpu.SemaphoreType.DMA,
        ],
    )
    def k(yt, ald, src, dst, zacc, accden_out,
          src_v, dst_v, rows, aldr, accden_s, sem0, sem1, isem0, isem1):
        cid = lax.axis_index("c")
        sid = lax.axis_index("s")
        wid = cid * NS + sid
        sems = (sem0, sem1)
        isems = (isem0, isem1)
        pltpu.sync_copy(zacc, accden_s.at[pl.ds(sid * rpt, rpt)])
        plsc.subcore_barrier()

        base = wid * epw

        def fetch_idx(t, b):
            off = base + t * C
            pltpu.async_copy(src.at[pl.ds(off, C)], src_v.at[b], isems[b])
            pltpu.async_copy(dst.at[pl.ds(off, C)], dst_v.at[b], isems[b])

        def drain_idx(b):
            pltpu.make_async_copy(src.at[pl.ds(0, C)], src_v.at[b],
                                  isems[b]).wait()
            pltpu.make_async_copy(dst.at[pl.ds(0, C)], dst_v.at[b],
                                  isems[b]).wait()

        def fetch_rows(b):
            pltpu.async_copy(yt.at[src_v.at[b]], rows.at[b], sems[b])
            pltpu.async_copy(ald.at[dst_v.at[b]], aldr.at[b], sems[b])

        def drain_rows(b):
            pltpu.make_async_copy(yt.at[src_v.at[b]], rows.at[b],
                                  sems[b]).wait()
            pltpu.make_async_copy(ald.at[dst_v.at[b]], aldr.at[b],
                                  sems[b]).wait()

        fetch_idx(0, 0)
        drain_idx(0)
        fetch_rows(0)
        fetch_idx(1, 1)

        def two(tt, carry):
            for b in range(2):
                t = tt * 2 + b
                drain_rows(b)
                drain_idx(1 - b)
                fetch_rows(1 - b)

                def edge(j, carry2):
                    t9 = rows[b, j, pl.ds(128, 16)] + aldr[b, j, :]
                    t9 = jnp.maximum(t9, t9 * 0.2)
                    ex = jnp.exp(t9)
                    rows[b, j, pl.ds(128, 16)] = ex
                    for kk in range(8):
                        s = ex[smap[kk]]
                        rows[b, j, pl.ds(kk * 16, 16)] = (
                            rows[b, j, pl.ds(kk * 16, 16)] * s)
                    return carry2

                lax.fori_loop(0, C, edge, 0, unroll=12)
                pltpu.sync_copy(rows.at[b], accden_s.at[dst_v.at[b]],
                                add=True)
                fetch_idx(jnp.minimum(t + 2, nchunks - 1), b)
            return carry

        lax.fori_loop(0, nchunks // 2, two, 0)
        drain_rows(0)
        drain_idx(1)
        plsc.subcore_barrier()
        pltpu.sync_copy(accden_s.at[pl.ds(sid * rpt, rpt)],
                        accden_out.at[cid, pl.ds(sid * rpt, rpt)])

    return k


def _gather_kernel(n, nrows):
    bpw = nrows // NW
    mesh = plsc.VectorSubcoreMesh(core_axis_name="c", subcore_axis_name="s")

    @functools.partial(
        pl.kernel,
        out_type=jax.ShapeDtypeStruct((nrows, 128), jnp.float32),
        mesh=mesh,
        scratch_types=[
            pltpu.VMEM((bpw,), jnp.int32),
            pltpu.VMEM((bpw, 128), jnp.float32),
            pltpu.SemaphoreType.DMA,
        ],
    )
    def k(emb, uv, out, idx_v, rows_v, sem):
        wid = lax.axis_index("c") * NS + lax.axis_index("s")
        base = wid * bpw
        pltpu.sync_copy(uv.at[pl.ds(base, bpw)], idx_v)
        pltpu.async_copy(emb.at[idx_v], rows_v, sem).wait()
        pltpu.sync_copy(rows_v, out.at[pl.ds(base, bpw)])

    return k


def kernel(x, edge_index, edges, groups, W1, a1_src, a1_dst, b1, bn_w, bn_b,
           Wsk, bsk, W2, a2_src, a2_dst, b2, Hw1, Hb1, Hln_g, Hln_b,
           Hw2, Hb2, Hw3, Hb3):
    f32 = jnp.float32
    N, D = x.shape
    H = W1.shape[0]
    E = edge_index.shape[1]
    B = edges.shape[0]
    heads = a1_src.shape[0]
    oc1 = H // heads

    n1 = ((N + 1 + NS * 8 - 1) // (NS * 8)) * (NS * 8)
    etot = E + N
    epad = ((etot + 2 * NW * C - 1) // (2 * NW * C)) * (2 * NW * C)
    epw = epad // NW
    nchunks = epw // C

    sl = jnp.arange(N, dtype=jnp.int32)
    pad = jnp.full((epad - etot,), N, dtype=jnp.int32)
    srcp = jnp.concatenate([edge_index[0].astype(jnp.int32), sl, pad])
    dstp = jnp.concatenate([edge_index[1].astype(jnp.int32), sl, pad])

    W1r = W1.reshape(heads, oc1, D)
    v1s = jnp.einsum('hc,hcd->hd', a1_src, W1r)
    v1d = jnp.einsum('hc,hcd->hd', a1_dst, W1r)
    wc1t = jnp.concatenate([W1, v1s, v1d], axis=0).T

    v2s = a2_src @ W2
    v2d = a2_dst @ W2
    wc2t = jnp.concatenate(
        [W2, v2s, v2d, jnp.zeros((14, H), f32)], axis=0).T

    zacc = jnp.zeros((n1 // NS, 144), f32)

    y1 = _tc_matmul(x, wc1t)
    y1p = jnp.zeros((n1, 144), f32).at[:N].set(y1)
    ald1 = jnp.zeros((n1, 16), f32).at[:N, :heads].set(y1[:, 136:144])

    ek1 = _edge_kernel(n1, epw, nchunks, list(range(8)))
    ad1 = ek1(y1p, ald1, srcp, dstp, zacc)
    acc1, den1 = ad1[..., :128], ad1[..., 128:]

    s8 = (jnp.arange(128, dtype=jnp.int32)[None, :] // oc1
          == jnp.arange(heads, dtype=jnp.int32)[:, None]).astype(f32)
    y2 = pl.pallas_call(
        _mid_body,
        out_shape=jax.ShapeDtypeStruct((N, 144), f32),
    )(acc1[:, :N], den1[:, :N, :heads], x, s8, b1.reshape(1, H),
      bn_w.reshape(1, H), bn_b.reshape(1, H), Wsk.T, bsk.reshape(1, H),
      wc2t)

    y2p = jnp.zeros((n1, 144), f32).at[:N].set(y2)
    ald2 = jnp.zeros((n1, 16), f32).at[:N, 0].set(y2[:, 129])

    ek2 = _edge_kernel(n1, epw, nchunks, [0] * 8)
    ad2 = ek2(y2p, ald2, srcp, dstp, zacc)
    acc2, den2 = ad2[..., :128], ad2[..., 128:]

    emb = pl.pallas_call(
        _emb_body,
        out_shape=jax.ShapeDtypeStruct((N, H), f32),
    )(acc2[:, :N], den2[:, :N, :1], b2.reshape(1, H))

    uv = jnp.concatenate([edges[:, 0], edges[:, 1]]).astype(jnp.int32)
    euv = _gather_kernel(N, 2 * B)(emb, uv)
    e = jnp.concatenate([euv[:B], euv[B:]], axis=1)

    out3 = pl.pallas_call(
        _heads_body,
        out_shape=jax.ShapeDtypeStruct((B, 3), f32),
    )(e, jnp.transpose(Hw1, (0, 2, 1)), Hb1[:, None, :], Hln_g[:, None, :],
      Hln_b[:, None, :], jnp.transpose(Hw2, (0, 2, 1)), Hb2[:, None, :],
      jnp.transpose(Hw3, (0, 2, 1)), Hb3[:, None, :])

    sel = out3[jnp.arange(B), groups]
    order = jnp.argsort(groups)
    return sel[order]
